# Initial kernel scaffold; baseline (speedup 1.0000x reference)
import jax
import jax.numpy as jnp
from jax.experimental import pallas as pl


def kernel(node_feat, edge_attr, edge_rsh, edge_index, Ws1, bs1, Ws2, bs2, Wr1, br1, Wr2, br2, Wl_s, bl_s, Wl_v):
    raise NotImplementedError("write your pallas kernel here")


# speedup vs baseline: 3.5979x; 3.5979x over previous
import functools

import jax
import jax.numpy as jnp
from jax import lax
from jax.experimental import pallas as pl
from jax.experimental.pallas import tpu as pltpu
from jax.experimental.pallas import tpu_sc as plsc

N = 10000
E = 160000
C = 128
H1 = 32
BXW = H1 + C
CH = 128
NCH = E // CH

_info = plsc.get_sparse_core_info()
_NC = _info.num_cores
_NS = _info.num_subcores
_NW = _NC * _NS
_RPT = 624
_TAIL = N - _RPT * _NS


def _gather(x, src, dst):
    mesh = plsc.VectorSubcoreMesh(core_axis_name="c", subcore_axis_name="s")

    @functools.partial(
        pl.kernel,
        mesh=mesh,
        out_type=(
            jax.ShapeDtypeStruct((E, C), jnp.float32),
            jax.ShapeDtypeStruct((E, C), jnp.float32),
        ),
        scratch_types=[
            pltpu.VMEM((CH,), jnp.int32),
            pltpu.VMEM((CH,), jnp.int32),
            pltpu.VMEM((CH, C), jnp.float32),
            pltpu.VMEM((CH, C), jnp.float32),
            pltpu.SemaphoreType.DMA,
            pltpu.SemaphoreType.DMA,
        ],
    )
    def gk(x_hbm, src_hbm, dst_hbm, xi_hbm, xj_hbm,
           si, di, xir, xjr, sem_a, sem_b):
        cid = lax.axis_index("c")
        sid = lax.axis_index("s")
        wid = sid * _NC + cid
        nch = 39 + jnp.where(wid < (NCH - 39 * _NW), 1, 0)

        def body(k, carry):
            off = (wid + k * _NW) * CH
            pltpu.sync_copy(src_hbm.at[pl.ds(off, CH)], si)
            pltpu.sync_copy(dst_hbm.at[pl.ds(off, CH)], di)
            ca = pltpu.async_copy(x_hbm.at[si], xir, sem_a)
            cb = pltpu.async_copy(x_hbm.at[di], xjr, sem_b)
            ca.wait()
            cb.wait()
            pltpu.sync_copy(xir, xi_hbm.at[pl.ds(off, CH)])
            pltpu.sync_copy(xjr, xj_hbm.at[pl.ds(off, CH)])
            return carry

        lax.fori_loop(0, nch, body, 0)

    return gk(x, src, dst)


def _edge_body(xi_ref, xj_ref, attr_ref, rsh_ref,
               wa_ref, wb_ref, bs1_ref, ws2_ref, bs2_ref,
               wr1_ref, br1_ref, wr2_ref, br2_ref,
               msg_ref):
    xi = xi_ref[...]
    xj = xj_ref[...]
    h = jax.nn.silu(
        jnp.dot(xi, wa_ref[...], preferred_element_type=jnp.float32)
        + jnp.dot(xj, wb_ref[...], preferred_element_type=jnp.float32)
        + bs1_ref[...])
    ws = jnp.dot(h, ws2_ref[...], preferred_element_type=jnp.float32) + bs2_ref[...]
    hr = jax.nn.silu(
        jnp.dot(attr_ref[...], wr1_ref[...], preferred_element_type=jnp.float32)
        + br1_ref[...])
    wr = jnp.dot(hr, wr2_ref[...], preferred_element_type=jnp.float32) + br2_ref[...]
    tw = ws * wr
    rsh = rsh_ref[...]
    mv = tw[:, C:] * xj
    msg_ref[0] = tw[:, :C] * xj * rsh[:, 0:1]
    msg_ref[1] = mv * rsh[:, 1:2]
    msg_ref[2] = mv * rsh[:, 2:3]
    msg_ref[3] = mv * rsh[:, 3:4]


def _edge_mlp(xi, xj, edge_attr, edge_rsh, Ws1, bs1, Ws2, bs2, Wr1, br1, Wr2, br2):
    BE = 2000
    WN = 2 * C
    return pl.pallas_call(
        _edge_body,
        grid=(E // BE,),
        in_specs=[
            pl.BlockSpec((BE, C), lambda i: (i, 0)),
            pl.BlockSpec((BE, C), lambda i: (i, 0)),
            pl.BlockSpec((BE, 16), lambda i: (i, 0)),
            pl.BlockSpec((BE, 4), lambda i: (i, 0)),
            pl.BlockSpec((C, H1), lambda i: (0, 0)),
            pl.BlockSpec((C, H1), lambda i: (0, 0)),
            pl.BlockSpec((1, H1), lambda i: (0, 0)),
            pl.BlockSpec((H1, WN), lambda i: (0, 0)),
            pl.BlockSpec((1, WN), lambda i: (0, 0)),
            pl.BlockSpec((16, H1), lambda i: (0, 0)),
            pl.BlockSpec((1, H1), lambda i: (0, 0)),
            pl.BlockSpec((H1, WN), lambda i: (0, 0)),
            pl.BlockSpec((1, WN), lambda i: (0, 0)),
        ],
        out_specs=pl.BlockSpec((4, BE, C), lambda i: (0, i, 0)),
        out_shape=jax.ShapeDtypeStruct((4, E, C), jnp.float32),
    )(xi, xj, edge_attr, edge_rsh,
      Ws1[:C], Ws1[C:], bs1.reshape(1, H1), Ws2, bs2.reshape(1, WN),
      Wr1, br1.reshape(1, H1), Wr2, br2.reshape(1, WN))


def _scatter_add(msg4, src, zeros):
    mesh = plsc.VectorSubcoreMesh(core_axis_name="c", subcore_axis_name="s")

    @functools.partial(
        pl.kernel,
        mesh=mesh,
        out_type=jax.ShapeDtypeStruct((4, N, C), jnp.float32),
        scratch_types=[
            pltpu.VMEM((1, CH), jnp.int32),
            pltpu.VMEM((CH, C), jnp.float32),
            pltpu.VMEM_SHARED((N, C), jnp.float32),
        ],
    )
    def sk(msg_hbm, src_hbm, zeros_hbm, accu_hbm, idx2, buf, acc_sh):
        cid = lax.axis_index("c")
        sid = lax.axis_index("s")
        rbase = sid * _RPT
        for gi in range(2):
            g = cid * 2 + gi
            pltpu.sync_copy(zeros_hbm.at[pl.ds(rbase, _RPT)],
                            acc_sh.at[pl.ds(rbase, _RPT)])

            @pl.when(sid == _NS - 1)
            def _zero_tail():
                pltpu.sync_copy(zeros_hbm.at[pl.ds(_RPT * _NS, _TAIL)],
                                acc_sh.at[pl.ds(_RPT * _NS, _TAIL)])

            plsc.subcore_barrier()
            nch = 78 + jnp.where(sid < (NCH - 78 * _NS), 1, 0)

            def body(k, carry):
                off = (sid + k * _NS) * CH
                pltpu.sync_copy(src_hbm.at[pl.ds(off, CH)], idx2.at[0])
                pltpu.sync_copy(msg_hbm.at[g, pl.ds(off, CH), :], buf)
                pltpu.sync_copy(buf, acc_sh.at[idx2.at[0]], add=True)
                return carry

            lax.fori_loop(0, nch, body, 0)
            plsc.subcore_barrier()
            pltpu.sync_copy(acc_sh.at[pl.ds(rbase, _RPT)],
                            accu_hbm.at[g, pl.ds(rbase, _RPT), :])

            @pl.when(sid == _NS - 1)
            def _write_tail():
                pltpu.sync_copy(acc_sh.at[pl.ds(_RPT * _NS, _TAIL)],
                                accu_hbm.at[g, pl.ds(_RPT * _NS, _TAIL), :])

            plsc.subcore_barrier()

    return sk(msg4, src, zeros)


def _final_body(accu_ref, wls_ref, bls_ref, wlv_ref, outs_ref, outv_ref):
    outs_ref[...] = (
        jnp.dot(accu_ref[0], wls_ref[...], preferred_element_type=jnp.float32)
        + bls_ref[...])
    for i in range(3):
        outv_ref[i] = jnp.dot(accu_ref[1 + i], wlv_ref[...],
                              preferred_element_type=jnp.float32)


def _final(accu, Wl_s, bl_s, Wl_v):
    BN = 2000
    return pl.pallas_call(
        _final_body,
        grid=(N // BN,),
        in_specs=[
            pl.BlockSpec((4, BN, C), lambda i: (0, i, 0)),
            pl.BlockSpec((C, C), lambda i: (0, 0)),
            pl.BlockSpec((1, C), lambda i: (0, 0)),
            pl.BlockSpec((C, C), lambda i: (0, 0)),
        ],
        out_specs=[
            pl.BlockSpec((BN, C), lambda i: (i, 0)),
            pl.BlockSpec((3, BN, C), lambda i: (0, i, 0)),
        ],
        out_shape=[
            jax.ShapeDtypeStruct((N, C), jnp.float32),
            jax.ShapeDtypeStruct((3, N, C), jnp.float32),
        ],
    )(accu, Wl_s, bl_s.reshape(1, C), Wl_v)


def kernel(node_feat, edge_attr, edge_rsh, edge_index, Ws1, bs1, Ws2, bs2,
           Wr1, br1, Wr2, br2, Wl_s, bl_s, Wl_v):
    src = edge_index[0]
    dst = edge_index[1]
    xi, xj = _gather(node_feat, src, dst)
    msg4 = _edge_mlp(xi, xj, edge_attr, edge_rsh,
                     Ws1, bs1, Ws2, bs2, Wr1, br1, Wr2, br2)
    zeros = jnp.zeros((N, C), jnp.float32)
    accu = _scatter_add(msg4, src, zeros)
    out_s, out_v = _final(accu, Wl_s, bl_s, Wl_v)
    out_vec = jnp.transpose(out_v, (1, 2, 0)).reshape(N, 3 * C)
    return jnp.concatenate([out_s, out_vec], axis=-1)
